# baseline (speedup 1.0000x reference)
import jax
import jax.numpy as jnp
from jax.experimental import pallas as pl
from jax.experimental.pallas import tpu as pltpu

IN_F = 768
OUT_F = 768
RANK = 8
N_EXP = 4
N_TOK = 4 * 2048

A_TILE = 1024
NA = N_TOK // (2 * A_TILE)
B_TILE = 1024
NB = N_TOK // B_TILE
HALF = N_TOK // 2


def _body(xlo_ref, xhi_ref, r_ref, wt_ref, d_ref, u_ref, o_ref,
          x16_ref, wc_ref, s_ref):
    p = pl.program_id(0)

    @pl.when(p == 0)
    def _init():
        s_ref[...] = jnp.zeros_like(s_ref)

    @pl.when(p < NA)
    def _phase_a():
        lo = xlo_ref[...]
        hi = xhi_ref[...]
        s_ref[...] += (jnp.sum(lo, axis=0, keepdims=True)
                       + jnp.sum(hi, axis=0, keepdims=True))
        x16_ref[pl.ds(p * A_TILE, A_TILE), :] = lo.astype(jnp.bfloat16)
        x16_ref[pl.ds(HALF + p * A_TILE, A_TILE), :] = hi.astype(jnp.bfloat16)

    @pl.when(p == NA)
    def _combine():
        om = jnp.dot(s_ref[...] * (1.0 / N_TOK), r_ref[...],
                     preferred_element_type=jnp.float32)
        o0, o1, o2, o3 = om[0, 0], om[0, 1], om[0, 2], om[0, 3]
        mx = jnp.maximum(jnp.maximum(o0, o1), jnp.maximum(o2, o3))
        e0 = jnp.exp(o0 - mx)
        e1 = jnp.exp(o1 - mx)
        e2 = jnp.exp(o2 - mx)
        e3 = jnp.exp(o3 - mx)
        z = e0 + e1 + e2 + e3
        idx = jax.lax.broadcasted_iota(jnp.int32, (1, N_EXP * RANK), 1) // RANK
        gcol = jnp.where(idx == 0, e0,
                         jnp.where(idx == 1, e1,
                                   jnp.where(idx == 2, e2, e3))) / z
        wc = wt_ref[...] + jnp.dot(
            d_ref[...] * gcol, u_ref[...], preferred_element_type=jnp.float32)
        wc_ref[...] = wc.astype(jnp.bfloat16)

    @pl.when(p >= NA)
    def _phase_b():
        j = p - NA
        o_ref[...] = jnp.dot(
            x16_ref[pl.ds(j * B_TILE, B_TILE), :], wc_ref[...],
            preferred_element_type=jnp.float32)


@jax.jit
def _run(x2, route_all, wt, dcat, ucat):
    return pl.pallas_call(
        _body,
        grid=(NA + NB,),
        in_specs=[
            pl.BlockSpec((A_TILE, IN_F),
                         lambda i: (jnp.minimum(i, NA - 1), 0)),
            pl.BlockSpec((A_TILE, IN_F),
                         lambda i: (jnp.minimum(i, NA - 1) + NA, 0)),
            pl.BlockSpec(route_all.shape, lambda i: (0, 0)),
            pl.BlockSpec(wt.shape, lambda i: (0, 0)),
            pl.BlockSpec(dcat.shape, lambda i: (0, 0)),
            pl.BlockSpec(ucat.shape, lambda i: (0, 0)),
        ],
        out_specs=pl.BlockSpec((B_TILE, OUT_F),
                               lambda i: (jnp.maximum(i - NA, 0), 0)),
        out_shape=jax.ShapeDtypeStruct((N_TOK, OUT_F), jnp.float32),
        scratch_shapes=[
            pltpu.VMEM((N_TOK, IN_F), jnp.bfloat16),
            pltpu.VMEM((IN_F, OUT_F), jnp.bfloat16),
            pltpu.VMEM((1, IN_F), jnp.float32),
        ],
    )(x2, x2, route_all, wt, dcat, ucat)


def kernel(input, task_id, W, lora_down, lora_up, lora_route):
    B, S, F = input.shape
    x2 = input.reshape(B * S, F)
    route_all = lora_route[0] + lora_route[1] + lora_route[2]
    wt = W.T
    dcat = jnp.transpose(lora_down[:N_EXP], (1, 0, 2)).reshape(F, N_EXP * RANK)
    ucat = lora_up[:N_EXP].reshape(N_EXP * RANK, OUT_F)
    out = _run(x2, route_all, wt, dcat, ucat)
    return out.reshape(B, S, OUT_F)


# speedup vs baseline: 1.0004x; 1.0004x over previous
import jax
import jax.numpy as jnp
from jax.experimental import pallas as pl
from jax.experimental.pallas import tpu as pltpu

IN_F = 768
OUT_F = 768
RANK = 8
N_EXP = 4
N_TOK = 4 * 2048

A_TILE = 2048
NA = N_TOK // (2 * A_TILE)
B_TILE = 2048
NB = N_TOK // B_TILE
HALF = N_TOK // 2


def _body(xlo_ref, xhi_ref, r_ref, wt_ref, d_ref, u_ref, o_ref,
          x16_ref, wc_ref, s_ref):
    p = pl.program_id(0)

    @pl.when(p == 0)
    def _init():
        s_ref[...] = jnp.zeros_like(s_ref)

    @pl.when(p < NA)
    def _phase_a():
        lo = xlo_ref[...]
        hi = xhi_ref[...]
        s_ref[...] += (jnp.sum(lo, axis=0, keepdims=True)
                       + jnp.sum(hi, axis=0, keepdims=True))
        x16_ref[pl.ds(p * A_TILE, A_TILE), :] = lo.astype(jnp.bfloat16)
        x16_ref[pl.ds(HALF + p * A_TILE, A_TILE), :] = hi.astype(jnp.bfloat16)

    @pl.when(p == NA)
    def _combine():
        om = jnp.dot(s_ref[...] * (1.0 / N_TOK), r_ref[...],
                     preferred_element_type=jnp.float32)
        o0, o1, o2, o3 = om[0, 0], om[0, 1], om[0, 2], om[0, 3]
        mx = jnp.maximum(jnp.maximum(o0, o1), jnp.maximum(o2, o3))
        e0 = jnp.exp(o0 - mx)
        e1 = jnp.exp(o1 - mx)
        e2 = jnp.exp(o2 - mx)
        e3 = jnp.exp(o3 - mx)
        z = e0 + e1 + e2 + e3
        idx = jax.lax.broadcasted_iota(jnp.int32, (1, N_EXP * RANK), 1) // RANK
        gcol = jnp.where(idx == 0, e0,
                         jnp.where(idx == 1, e1,
                                   jnp.where(idx == 2, e2, e3))) / z
        wc = wt_ref[...] + jnp.dot(
            d_ref[...] * gcol, u_ref[...], preferred_element_type=jnp.float32)
        wc_ref[...] = wc.astype(jnp.bfloat16)

    @pl.when(p >= NA)
    def _phase_b():
        j = p - NA
        o_ref[...] = jnp.dot(
            x16_ref[pl.ds(j * B_TILE, B_TILE), :], wc_ref[...],
            preferred_element_type=jnp.float32)


@jax.jit
def _run(x2, route_all, wt, dcat, ucat):
    return pl.pallas_call(
        _body,
        grid=(NA + NB,),
        in_specs=[
            pl.BlockSpec((A_TILE, IN_F),
                         lambda i: (jnp.minimum(i, NA - 1), 0)),
            pl.BlockSpec((A_TILE, IN_F),
                         lambda i: (jnp.minimum(i, NA - 1) + NA, 0)),
            pl.BlockSpec(route_all.shape, lambda i: (0, 0)),
            pl.BlockSpec(wt.shape, lambda i: (0, 0)),
            pl.BlockSpec(dcat.shape, lambda i: (0, 0)),
            pl.BlockSpec(ucat.shape, lambda i: (0, 0)),
        ],
        out_specs=pl.BlockSpec((B_TILE, OUT_F),
                               lambda i: (jnp.maximum(i - NA, 0), 0)),
        out_shape=jax.ShapeDtypeStruct((N_TOK, OUT_F), jnp.float32),
        scratch_shapes=[
            pltpu.VMEM((N_TOK, IN_F), jnp.bfloat16),
            pltpu.VMEM((IN_F, OUT_F), jnp.bfloat16),
            pltpu.VMEM((1, IN_F), jnp.float32),
        ],
    )(x2, x2, route_all, wt, dcat, ucat)


def kernel(input, task_id, W, lora_down, lora_up, lora_route):
    B, S, F = input.shape
    x2 = input.reshape(B * S, F)
    route_all = lora_route[0] + lora_route[1] + lora_route[2]
    wt = W.T
    dcat = jnp.transpose(lora_down[:N_EXP], (1, 0, 2)).reshape(F, N_EXP * RANK)
    ucat = lora_up[:N_EXP].reshape(N_EXP * RANK, OUT_F)
    out = _run(x2, route_all, wt, dcat, ucat)
    return out.reshape(B, S, OUT_F)


# speedup vs baseline: 1.0214x; 1.0210x over previous
import jax
import jax.numpy as jnp
from jax.experimental import pallas as pl
from jax.experimental.pallas import tpu as pltpu

IN_F = 768
OUT_F = 768
RANK = 8
N_EXP = 4
N_TOK = 4 * 2048

ROW_TILE = 2048
NT = N_TOK // ROW_TILE


def _fused_body(x_ref, r_ref, wt_ref, d_ref, u_ref, o_ref,
                x16_ref, wc_ref, s_ref):
    p = pl.program_id(0)

    @pl.when(p == 0)
    def _init():
        s_ref[...] = jnp.zeros_like(s_ref)

    @pl.when(p < NT)
    def _phase_a():
        xb = x_ref[...]
        s_ref[...] += jnp.sum(xb, axis=0, keepdims=True)
        x16_ref[pl.ds(p * ROW_TILE, ROW_TILE), :] = xb.astype(jnp.bfloat16)

    @pl.when(p == NT)
    def _combine():
        om = jnp.dot(s_ref[...] * (1.0 / N_TOK), r_ref[...],
                     preferred_element_type=jnp.float32)
        o0, o1, o2, o3 = om[0, 0], om[0, 1], om[0, 2], om[0, 3]
        mx = jnp.maximum(jnp.maximum(o0, o1), jnp.maximum(o2, o3))
        e0 = jnp.exp(o0 - mx)
        e1 = jnp.exp(o1 - mx)
        e2 = jnp.exp(o2 - mx)
        e3 = jnp.exp(o3 - mx)
        z = e0 + e1 + e2 + e3
        idx = jax.lax.broadcasted_iota(jnp.int32, (1, N_EXP * RANK), 1) // RANK
        gcol = jnp.where(idx == 0, e0,
                         jnp.where(idx == 1, e1,
                                   jnp.where(idx == 2, e2, e3))) / z
        wc = wt_ref[...] + jnp.dot(
            d_ref[...] * gcol, u_ref[...], preferred_element_type=jnp.float32)
        wc_ref[...] = wc.astype(jnp.bfloat16)

    @pl.when(p >= NT)
    def _phase_b():
        j = p - NT
        o_ref[...] = jnp.dot(
            x16_ref[pl.ds(j * ROW_TILE, ROW_TILE), :], wc_ref[...],
            preferred_element_type=jnp.float32)


@jax.jit
def _run(x2, route_all, wt, dcat, ucat):
    return pl.pallas_call(
        _fused_body,
        grid=(2 * NT,),
        in_specs=[
            pl.BlockSpec((ROW_TILE, IN_F),
                         lambda i: (jnp.minimum(i, NT - 1), 0)),
            pl.BlockSpec(route_all.shape, lambda i: (0, 0)),
            pl.BlockSpec(wt.shape, lambda i: (0, 0)),
            pl.BlockSpec(dcat.shape, lambda i: (0, 0)),
            pl.BlockSpec(ucat.shape, lambda i: (0, 0)),
        ],
        out_specs=pl.BlockSpec((ROW_TILE, OUT_F),
                               lambda i: (jnp.maximum(i - NT, 0), 0)),
        out_shape=jax.ShapeDtypeStruct((N_TOK, OUT_F), jnp.float32),
        scratch_shapes=[
            pltpu.VMEM((N_TOK, IN_F), jnp.bfloat16),
            pltpu.VMEM((IN_F, OUT_F), jnp.bfloat16),
            pltpu.VMEM((1, IN_F), jnp.float32),
        ],
    )(x2, route_all, wt, dcat, ucat)


def kernel(input, task_id, W, lora_down, lora_up, lora_route):
    B, S, F = input.shape
    x2 = input.reshape(B * S, F)
    route_all = lora_route[0] + lora_route[1] + lora_route[2]
    wt = W.T
    dcat = jnp.transpose(lora_down[:N_EXP], (1, 0, 2)).reshape(F, N_EXP * RANK)
    ucat = lora_up[:N_EXP].reshape(N_EXP * RANK, OUT_F)
    out = _run(x2, route_all, wt, dcat, ucat)
    return out.reshape(B, S, OUT_F)


# speedup vs baseline: 1.0222x; 1.0008x over previous
import jax
import jax.numpy as jnp
from jax.experimental import pallas as pl
from jax.experimental.pallas import tpu as pltpu

IN_F = 768
OUT_F = 768
RANK = 8
N_EXP = 4
N_TOK = 4 * 2048

ROW_TILE = 2048
NT = N_TOK // ROW_TILE


def _fused_body(x_ref, r_ref, wt_ref, d_ref, u_ref, o_ref,
                x16_ref, wc_ref, s_ref):
    p = pl.program_id(0)

    @pl.when(p == 0)
    def _init():
        s_ref[...] = jnp.zeros_like(s_ref)

    @pl.when(p < NT)
    def _phase_a():
        xb = x_ref[...]
        s_ref[...] += jnp.sum(xb, axis=0, keepdims=True)
        x16_ref[pl.ds(p * ROW_TILE, ROW_TILE), :] = xb.astype(jnp.bfloat16)

    @pl.when(p == NT)
    def _combine():
        om = jnp.dot(s_ref[...] * (1.0 / N_TOK), r_ref[...],
                     preferred_element_type=jnp.float32)
        o0, o1, o2, o3 = om[0, 0], om[0, 1], om[0, 2], om[0, 3]
        mx = jnp.maximum(jnp.maximum(o0, o1), jnp.maximum(o2, o3))
        e0 = jnp.exp(o0 - mx)
        e1 = jnp.exp(o1 - mx)
        e2 = jnp.exp(o2 - mx)
        e3 = jnp.exp(o3 - mx)
        z = e0 + e1 + e2 + e3
        idx = jax.lax.broadcasted_iota(jnp.int32, (1, N_EXP * RANK), 1) // RANK
        gcol = jnp.where(idx == 0, e0,
                         jnp.where(idx == 1, e1,
                                   jnp.where(idx == 2, e2, e3))) / z
        wc = wt_ref[...] + jnp.dot(
            d_ref[...] * gcol, u_ref[...], preferred_element_type=jnp.float32)
        wc_ref[...] = wc.astype(jnp.bfloat16)

    @pl.when(p >= NT)
    def _phase_b():
        j = p - NT
        half = ROW_TILE // 2
        o_ref[0:half, :] = jnp.dot(
            x16_ref[pl.ds(j * ROW_TILE, half), :], wc_ref[...],
            preferred_element_type=jnp.float32)
        o_ref[half:ROW_TILE, :] = jnp.dot(
            x16_ref[pl.ds(j * ROW_TILE + half, half), :], wc_ref[...],
            preferred_element_type=jnp.float32)


@jax.jit
def _run(x2, route_all, wt, dcat, ucat):
    return pl.pallas_call(
        _fused_body,
        grid=(2 * NT,),
        in_specs=[
            pl.BlockSpec((ROW_TILE, IN_F),
                         lambda i: (jnp.minimum(i, NT - 1), 0)),
            pl.BlockSpec(route_all.shape, lambda i: (0, 0)),
            pl.BlockSpec(wt.shape, lambda i: (0, 0)),
            pl.BlockSpec(dcat.shape, lambda i: (0, 0)),
            pl.BlockSpec(ucat.shape, lambda i: (0, 0)),
        ],
        out_specs=pl.BlockSpec((ROW_TILE, OUT_F),
                               lambda i: (jnp.maximum(i - NT, 0), 0)),
        out_shape=jax.ShapeDtypeStruct((N_TOK, OUT_F), jnp.float32),
        scratch_shapes=[
            pltpu.VMEM((N_TOK, IN_F), jnp.bfloat16),
            pltpu.VMEM((IN_F, OUT_F), jnp.bfloat16),
            pltpu.VMEM((1, IN_F), jnp.float32),
        ],
    )(x2, route_all, wt, dcat, ucat)


def kernel(input, task_id, W, lora_down, lora_up, lora_route):
    B, S, F = input.shape
    x2 = input.reshape(B * S, F)
    route_all = lora_route[0] + lora_route[1] + lora_route[2]
    wt = W.T
    dcat = jnp.transpose(lora_down[:N_EXP], (1, 0, 2)).reshape(F, N_EXP * RANK)
    ucat = lora_up[:N_EXP].reshape(N_EXP * RANK, OUT_F)
    out = _run(x2, route_all, wt, dcat, ucat)
    return out.reshape(B, S, OUT_F)


# speedup vs baseline: 1.0463x; 1.0236x over previous
import jax
import jax.numpy as jnp
from jax.experimental import pallas as pl
from jax.experimental.pallas import tpu as pltpu

IN_F = 768
OUT_F = 768
RANK = 8
N_EXP = 4
N_TOK = 4 * 2048

ROW_TILE = 2048
NT = N_TOK // ROW_TILE


def _fused_body(x_ref, r_ref, wt_ref, d_ref, u_ref, o_ref,
                x16_ref, wc_ref, s_ref):
    p = pl.program_id(0)

    @pl.when(p == 0)
    def _init():
        s_ref[...] = jnp.zeros_like(s_ref)

    @pl.when(p < NT)
    def _phase_a():
        xb = x_ref[...]
        s_ref[...] += jnp.sum(xb, axis=0, keepdims=True)
        x16_ref[pl.ds(p * ROW_TILE, ROW_TILE), :] = xb.astype(jnp.bfloat16)

    @pl.when(p == NT)
    def _combine():
        om = jnp.dot(s_ref[...] * (1.0 / N_TOK), r_ref[...],
                     preferred_element_type=jnp.float32)
        o0, o1, o2, o3 = om[0, 0], om[0, 1], om[0, 2], om[0, 3]
        mx = jnp.maximum(jnp.maximum(o0, o1), jnp.maximum(o2, o3))
        e0 = jnp.exp(o0 - mx)
        e1 = jnp.exp(o1 - mx)
        e2 = jnp.exp(o2 - mx)
        e3 = jnp.exp(o3 - mx)
        z = e0 + e1 + e2 + e3
        idx = jax.lax.broadcasted_iota(jnp.int32, (1, N_EXP * RANK), 1) // RANK
        gcol = jnp.where(idx == 0, e0,
                         jnp.where(idx == 1, e1,
                                   jnp.where(idx == 2, e2, e3))) / z
        delta = jnp.dot(
            d_ref[...] * gcol, u_ref[...], preferred_element_type=jnp.float32)
        wc_ref[...] = wt_ref[...] + delta.astype(jnp.bfloat16)

    @pl.when(p >= NT)
    def _phase_b():
        j = p - NT
        o_ref[...] = jnp.dot(
            x16_ref[pl.ds(j * ROW_TILE, ROW_TILE), :], wc_ref[...],
            preferred_element_type=jnp.float32)


@jax.jit
def _run(x2, route_all, wt, dcat, ucat):
    return pl.pallas_call(
        _fused_body,
        grid=(2 * NT,),
        in_specs=[
            pl.BlockSpec((ROW_TILE, IN_F),
                         lambda i: (jnp.minimum(i, NT - 1), 0)),
            pl.BlockSpec(route_all.shape, lambda i: (0, 0)),
            pl.BlockSpec(wt.shape, lambda i: (0, 0)),
            pl.BlockSpec(dcat.shape, lambda i: (0, 0)),
            pl.BlockSpec(ucat.shape, lambda i: (0, 0)),
        ],
        out_specs=pl.BlockSpec((ROW_TILE, OUT_F),
                               lambda i: (jnp.maximum(i - NT, 0), 0)),
        out_shape=jax.ShapeDtypeStruct((N_TOK, OUT_F), jnp.float32),
        scratch_shapes=[
            pltpu.VMEM((N_TOK, IN_F), jnp.bfloat16),
            pltpu.VMEM((IN_F, OUT_F), jnp.bfloat16),
            pltpu.VMEM((1, IN_F), jnp.float32),
        ],
    )(x2, route_all, wt, dcat, ucat)


def kernel(input, task_id, W, lora_down, lora_up, lora_route):
    B, S, F = input.shape
    x2 = input.reshape(B * S, F)
    route_all = lora_route[0] + lora_route[1] + lora_route[2]
    wt = W.T.astype(jnp.bfloat16)
    dcat = jnp.transpose(lora_down[:N_EXP], (1, 0, 2)).reshape(F, N_EXP * RANK)
    ucat = lora_up[:N_EXP].reshape(N_EXP * RANK, OUT_F)
    out = _run(x2, route_all, wt, dcat, ucat)
    return out.reshape(B, S, OUT_F)


# speedup vs baseline: 1.0506x; 1.0041x over previous
import jax
import jax.numpy as jnp
from jax.experimental import pallas as pl
from jax.experimental.pallas import tpu as pltpu

IN_F = 768
OUT_F = 768
RANK = 8
N_EXP = 4
N_TOK = 4 * 2048

ROW_TILE = 2048
NT = N_TOK // ROW_TILE


def _fused_body(x_ref, r_ref, wt_ref, d_ref, u_ref, o_ref,
                x16_ref, wc_ref, s_ref):
    p = pl.program_id(0)

    @pl.when(p == 0)
    def _init():
        s_ref[...] = jnp.zeros_like(s_ref)

    @pl.when(p < NT)
    def _phase_a():
        xb = x_ref[...]
        s_ref[...] += jnp.sum(xb, axis=0, keepdims=True)
        x16_ref[pl.ds(p * ROW_TILE, ROW_TILE), :] = xb.astype(jnp.bfloat16)

    @pl.when(p == NT)
    def _combine():
        om = jnp.dot(s_ref[...] * (1.0 / N_TOK), r_ref[...],
                     preferred_element_type=jnp.float32)
        o0, o1, o2, o3 = om[0, 0], om[0, 1], om[0, 2], om[0, 3]
        mx = jnp.maximum(jnp.maximum(o0, o1), jnp.maximum(o2, o3))
        e0 = jnp.exp(o0 - mx)
        e1 = jnp.exp(o1 - mx)
        e2 = jnp.exp(o2 - mx)
        e3 = jnp.exp(o3 - mx)
        z = e0 + e1 + e2 + e3
        idx = jax.lax.broadcasted_iota(jnp.int32, (1, N_EXP * RANK), 1) // RANK
        gcol = jnp.where(idx == 0, e0,
                         jnp.where(idx == 1, e1,
                                   jnp.where(idx == 2, e2, e3))) / z
        dg16 = (d_ref[...] * gcol).astype(jnp.bfloat16)
        delta = jnp.dot(dg16, u_ref[...], preferred_element_type=jnp.float32)
        wc_ref[...] = wt_ref[...] + delta.astype(jnp.bfloat16)

    @pl.when(p >= NT)
    def _phase_b():
        j = p - NT
        o_ref[...] = jnp.dot(
            x16_ref[pl.ds(j * ROW_TILE, ROW_TILE), :], wc_ref[...],
            preferred_element_type=jnp.float32)


@jax.jit
def _run(x2, route_all, wt, dcat, ucat):
    return pl.pallas_call(
        _fused_body,
        grid=(2 * NT,),
        in_specs=[
            pl.BlockSpec((ROW_TILE, IN_F),
                         lambda i: (jnp.minimum(i, NT - 1), 0)),
            pl.BlockSpec(route_all.shape, lambda i: (0, 0)),
            pl.BlockSpec(wt.shape, lambda i: (0, 0)),
            pl.BlockSpec(dcat.shape, lambda i: (0, 0)),
            pl.BlockSpec(ucat.shape, lambda i: (0, 0)),
        ],
        out_specs=pl.BlockSpec((ROW_TILE, OUT_F),
                               lambda i: (jnp.maximum(i - NT, 0), 0)),
        out_shape=jax.ShapeDtypeStruct((N_TOK, OUT_F), jnp.float32),
        scratch_shapes=[
            pltpu.VMEM((N_TOK, IN_F), jnp.bfloat16),
            pltpu.VMEM((IN_F, OUT_F), jnp.bfloat16),
            pltpu.VMEM((1, IN_F), jnp.float32),
        ],
    )(x2, route_all, wt, dcat, ucat)


def kernel(input, task_id, W, lora_down, lora_up, lora_route):
    B, S, F = input.shape
    x2 = input.reshape(B * S, F)
    route_all = lora_route[0] + lora_route[1] + lora_route[2]
    wt = W.T.astype(jnp.bfloat16)
    dcat = jnp.transpose(lora_down[:N_EXP], (1, 0, 2)).reshape(F, N_EXP * RANK)
    ucat = lora_up[:N_EXP].reshape(N_EXP * RANK, OUT_F).astype(jnp.bfloat16)
    out = _run(x2, route_all, wt, dcat, ucat)
    return out.reshape(B, S, OUT_F)
